# baseline (device time: 45147 ns/iter reference)
import os

import jax
import jax.numpy as jnp
from jax import lax
from jax.experimental import pallas as pl
from jax.experimental.pallas import tpu as pltpu

N_DEV = 4
N_EXPERTS = 32
CAP = 51
CAP_PAD = 64
E_LOCAL = N_EXPERTS // N_DEV
ROWS_PER_DEV = E_LOCAL * CAP_PAD
N_SUB = E_LOCAL // 2
N_WBUF = 4
N_ROWS = N_EXPERTS * CAP_PAD
BLK = 256
DROP_SENTINEL = 3000


def _moe_pallas(x, rt_col, rt_row, expert_W, tok_per_dev):
    n_tokens, d_model = x.shape
    _, _, h_out = expert_W.shape
    n_blk = n_tokens // BLK

    def body(x_ref, rtc_ref, rtr_ref, w_hbm, out_ref, table, xd,
             slot_col, slot_row, wbuf, w_sems,
             send_cw, recv_cw, send_ccw, recv_ccw):
        my = lax.axis_index("i")
        left = lax.rem(my + (N_DEV - 1), N_DEV)
        right = lax.rem(my + 1, N_DEV)
        base = my * ROWS_PER_DEV
        pipe = not os.environ.get("KERNEL_NO_RING")

        barrier_sem = pltpu.get_barrier_semaphore()
        for nbr in [left, right]:
            pl.semaphore_signal(
                barrier_sem, inc=1,
                device_id=(nbr,), device_id_type=pl.DeviceIdType.MESH,
            )
        pl.semaphore_wait(barrier_sem, 2)

        def w_copy(e):
            return pltpu.make_async_copy(
                w_hbm.at[e], wbuf.at[e % N_WBUF], w_sems.at[e % N_WBUF]
            )

        for e in range(N_WBUF):
            w_copy(e).start()

        def mk_cw(h, c, s):
            r0 = c * ROWS_PER_DEV + s * CAP_PAD
            return pltpu.make_async_remote_copy(
                src_ref=table.at[pl.ds(r0, CAP_PAD)],
                dst_ref=table.at[pl.ds(r0, CAP_PAD)],
                send_sem=send_cw.at[h, s],
                recv_sem=recv_cw.at[h, s],
                device_id=(right,),
                device_id_type=pl.DeviceIdType.MESH,
            )

        def mk_ccw(h, c, s):
            r0 = c * ROWS_PER_DEV + (N_SUB + s) * CAP_PAD
            return pltpu.make_async_remote_copy(
                src_ref=table.at[pl.ds(r0, CAP_PAD)],
                dst_ref=table.at[pl.ds(r0, CAP_PAD)],
                send_sem=send_ccw.at[h, s],
                recv_sem=recv_ccw.at[h, s],
                device_id=(left,),
                device_id_type=pl.DeviceIdType.MESH,
            )

        ir = lax.broadcasted_iota(jnp.int32, (BLK, BLK), 0)
        ic = lax.broadcasted_iota(jnp.int32, (BLK, BLK), 1)
        tril = (ir >= ic).astype(jnp.bfloat16)
        triu = (ir <= ic).astype(jnp.bfloat16)

        lane32 = lax.broadcasted_iota(jnp.int32, (BLK, N_EXPERTS), 1)
        prefix = jnp.zeros((1, N_EXPERTS), jnp.float32)
        for b in range(n_blk):
            e_b = rtc_ref[b * BLK:(b + 1) * BLK, :]
            ohm = e_b == lane32
            cum_b = jnp.dot(
                tril, ohm.astype(jnp.bfloat16),
                preferred_element_type=jnp.float32,
            ) + prefix
            pos_b = jnp.sum(
                cum_b * ohm.astype(jnp.float32), axis=1, keepdims=True
            ) - 1.0
            slot_f = jnp.where(
                pos_b < float(CAP),
                e_b.astype(jnp.float32) * float(CAP_PAD) + pos_b,
                float(DROP_SENTINEL),
            )
            slot_col[b * BLK:(b + 1) * BLK, :] = slot_f.astype(jnp.int32)
            prefix = cum_b[BLK - 1:BLK, :]

        sub32 = lax.broadcasted_iota(jnp.int32, (N_EXPERTS, BLK), 0)
        prefr = jnp.zeros((N_EXPERTS, 1), jnp.float32)
        for b in range(n_blk):
            e_rb = rtr_ref[:, b * BLK:(b + 1) * BLK]
            ohm = sub32 == e_rb
            cum_rb = jnp.dot(
                ohm.astype(jnp.bfloat16), triu,
                preferred_element_type=jnp.float32,
            ) + prefr
            pos_rb = jnp.sum(
                cum_rb * ohm.astype(jnp.float32), axis=0, keepdims=True
            ) - 1.0
            slot_rf = jnp.where(
                pos_rb < float(CAP),
                e_rb.astype(jnp.float32) * float(CAP_PAD) + pos_rb,
                float(DROP_SENTINEL),
            )
            slot_row[:, b * BLK:(b + 1) * BLK] = slot_rf.astype(jnp.int32)
            prefr = cum_rb[:, BLK - 1:BLK]

        row_ids = base + lax.broadcasted_iota(
            jnp.int32, (ROWS_PER_DEV, n_tokens), 0
        )
        disp = (slot_row[...] == row_ids).astype(jnp.bfloat16)
        xd[...] = jnp.dot(
            disp, x_ref[...].astype(jnp.bfloat16),
            preferred_element_type=jnp.float32,
        )

        for e in range(E_LOCAL):
            w_copy(e).wait()
            res = jnp.dot(
                xd[e * CAP_PAD:(e + 1) * CAP_PAD],
                wbuf[e % N_WBUF],
                preferred_element_type=jnp.float32,
            )
            table[pl.ds(base + e * CAP_PAD, CAP_PAD), :] = res.astype(
                jnp.bfloat16
            )
            if e + N_WBUF < E_LOCAL:
                w_copy(e + N_WBUF).start()
            if pipe:
                if e < N_SUB:
                    mk_cw(0, my, e).start()
                else:
                    mk_ccw(0, my, e - N_SUB).start()

        my_slot = slot_col[pl.ds(my * tok_per_dev, tok_per_dev), :]
        chunk_col_ids = lax.broadcasted_iota(
            jnp.int32, (tok_per_dev, ROWS_PER_DEV), 1
        )

        def combine(c, first):
            comb = (my_slot == c * ROWS_PER_DEV + chunk_col_ids).astype(
                jnp.bfloat16
            )
            part = jnp.dot(
                comb,
                table[pl.ds(c * ROWS_PER_DEV, ROWS_PER_DEV), :],
                preferred_element_type=jnp.float32,
            )
            out_ref[...] = part if first else out_ref[...] + part

        if pipe:
            combine(my, first=True)
            for h in range(1, N_DEV - 1):
                c_cw = lax.rem(my - h + N_DEV, N_DEV)
                c_ccw = lax.rem(my + h, N_DEV)
                for s in range(N_SUB):
                    mk_cw(h - 1, c_cw, s).wait_recv()
                    mk_cw(h, c_cw, s).start()
                    mk_ccw(h - 1, c_ccw, s).wait_recv()
                    mk_ccw(h, c_ccw, s).start()
                if h == N_DEV - 2:
                    combine(lax.rem(my + 2, N_DEV), first=False)
            h_last = N_DEV - 2
            c_cw = lax.rem(my + 1, N_DEV)
            c_ccw = lax.rem(my - 1 + N_DEV, N_DEV)
            for s in range(N_SUB):
                mk_cw(h_last, c_cw, s).wait_recv()
                mk_ccw(h_last, c_ccw, s).wait_recv()
            combine(c_cw, first=False)
            combine(c_ccw, first=False)
            for h in range(N_DEV - 1):
                c_cw = lax.rem(my - h + N_DEV, N_DEV)
                c_ccw = lax.rem(my + h, N_DEV)
                for s in range(N_SUB):
                    mk_cw(h, c_cw, s).wait_send()
                    mk_ccw(h, c_ccw, s).wait_send()
        else:
            combine(my, first=True)
            for d in range(1, N_DEV):
                combine(lax.rem(my + d, N_DEV), first=False)

    return pl.pallas_call(
        body,
        out_shape=jax.ShapeDtypeStruct((tok_per_dev, h_out), jnp.float32),
        in_specs=[
            pl.BlockSpec(memory_space=pltpu.VMEM),
            pl.BlockSpec(memory_space=pltpu.VMEM),
            pl.BlockSpec(memory_space=pltpu.VMEM),
            pl.BlockSpec(memory_space=pltpu.MemorySpace.HBM),
        ],
        out_specs=pl.BlockSpec(memory_space=pltpu.VMEM),
        scratch_shapes=[
            pltpu.VMEM((N_ROWS, h_out), jnp.bfloat16),
            pltpu.VMEM((ROWS_PER_DEV, d_model), jnp.float32),
            pltpu.VMEM((n_tokens, 1), jnp.int32),
            pltpu.VMEM((1, n_tokens), jnp.int32),
            pltpu.VMEM((N_WBUF, d_model, h_out), jnp.float32),
            pltpu.SemaphoreType.DMA((N_WBUF,)),
            pltpu.SemaphoreType.DMA((N_DEV - 1, N_SUB)),
            pltpu.SemaphoreType.DMA((N_DEV - 1, N_SUB)),
            pltpu.SemaphoreType.DMA((N_DEV - 1, N_SUB)),
            pltpu.SemaphoreType.DMA((N_DEV - 1, N_SUB)),
        ],
        compiler_params=pltpu.CompilerParams(
            collective_id=0,
            vmem_limit_bytes=100 * 1024 * 1024,
        ),
    )(x, rt_col, rt_row, expert_W)


def kernel(x, router_W, route_idx, expert_W):
    n_tokens, _ = x.shape
    del router_W
    tok_per_dev = n_tokens // N_DEV
    rt_row = jnp.transpose(route_idx)
    return _moe_pallas(x, route_idx, rt_row, expert_W, tok_per_dev)


# device time: 43632 ns/iter; 1.0347x vs baseline; 1.0347x over previous
import os

import jax
import jax.numpy as jnp
from jax import lax
from jax.experimental import pallas as pl
from jax.experimental.pallas import tpu as pltpu

N_DEV = 4
N_EXPERTS = 32
CAP = 51
CAP_PAD = 64
E_LOCAL = N_EXPERTS // N_DEV
ROWS_PER_DEV = E_LOCAL * CAP_PAD
N_SUB = E_LOCAL // 2
N_WBUF = 4
N_ROWS = N_EXPERTS * CAP_PAD
SEND_ROWS = 56
BLK = 256
DROP_SENTINEL = 3000


def _moe_pallas(x, rt_col, rt_row, expert_W, tok_per_dev):
    n_tokens, d_model = x.shape
    _, _, h_out = expert_W.shape
    n_blk = n_tokens // BLK

    def body(x_ref, rtc_ref, rtr_ref, w_hbm, out_ref, table, xd,
             slot_col, slot_row, wbuf, w_sems,
             send_cw, recv_cw, send_ccw, recv_ccw):
        my = lax.axis_index("i")
        left = lax.rem(my + (N_DEV - 1), N_DEV)
        right = lax.rem(my + 1, N_DEV)
        base = my * ROWS_PER_DEV
        pipe = not os.environ.get("KERNEL_NO_RING")

        table[...] = jnp.zeros((N_ROWS, h_out), jnp.bfloat16)

        barrier_sem = pltpu.get_barrier_semaphore()
        for nbr in [left, right]:
            pl.semaphore_signal(
                barrier_sem, inc=1,
                device_id=(nbr,), device_id_type=pl.DeviceIdType.MESH,
            )
        pl.semaphore_wait(barrier_sem, 2)

        def w_copy(e):
            return pltpu.make_async_copy(
                w_hbm.at[e], wbuf.at[e % N_WBUF], w_sems.at[e % N_WBUF]
            )

        for e in range(N_WBUF):
            w_copy(e).start()

        def mk_cw(h, c, s):
            r0 = c * ROWS_PER_DEV + s * CAP_PAD
            return pltpu.make_async_remote_copy(
                src_ref=table.at[pl.ds(r0, SEND_ROWS)],
                dst_ref=table.at[pl.ds(r0, SEND_ROWS)],
                send_sem=send_cw.at[h, s],
                recv_sem=recv_cw.at[h, s],
                device_id=(right,),
                device_id_type=pl.DeviceIdType.MESH,
            )

        def mk_ccw(h, c, s):
            r0 = c * ROWS_PER_DEV + (N_SUB + s) * CAP_PAD
            return pltpu.make_async_remote_copy(
                src_ref=table.at[pl.ds(r0, SEND_ROWS)],
                dst_ref=table.at[pl.ds(r0, SEND_ROWS)],
                send_sem=send_ccw.at[h, s],
                recv_sem=recv_ccw.at[h, s],
                device_id=(left,),
                device_id_type=pl.DeviceIdType.MESH,
            )

        ir = lax.broadcasted_iota(jnp.int32, (BLK, BLK), 0)
        ic = lax.broadcasted_iota(jnp.int32, (BLK, BLK), 1)
        tril = (ir >= ic).astype(jnp.bfloat16)
        triu = (ir <= ic).astype(jnp.bfloat16)

        lane32 = lax.broadcasted_iota(jnp.int32, (BLK, N_EXPERTS), 1)
        prefix = jnp.zeros((1, N_EXPERTS), jnp.float32)
        for b in range(n_blk):
            e_b = rtc_ref[b * BLK:(b + 1) * BLK, :]
            ohm = e_b == lane32
            cum_b = jnp.dot(
                tril, ohm.astype(jnp.bfloat16),
                preferred_element_type=jnp.float32,
            ) + prefix
            pos_b = jnp.sum(
                cum_b * ohm.astype(jnp.float32), axis=1, keepdims=True
            ) - 1.0
            slot_f = jnp.where(
                pos_b < float(CAP),
                e_b.astype(jnp.float32) * float(CAP_PAD) + pos_b,
                float(DROP_SENTINEL),
            )
            slot_col[b * BLK:(b + 1) * BLK, :] = slot_f.astype(jnp.int32)
            prefix = cum_b[BLK - 1:BLK, :]

        sub32 = lax.broadcasted_iota(jnp.int32, (N_EXPERTS, BLK), 0)
        prefr = jnp.zeros((N_EXPERTS, 1), jnp.float32)
        for b in range(n_blk):
            e_rb = rtr_ref[:, b * BLK:(b + 1) * BLK]
            ohm = sub32 == e_rb
            cum_rb = jnp.dot(
                ohm.astype(jnp.bfloat16), triu,
                preferred_element_type=jnp.float32,
            ) + prefr
            pos_rb = jnp.sum(
                cum_rb * ohm.astype(jnp.float32), axis=0, keepdims=True
            ) - 1.0
            slot_rf = jnp.where(
                pos_rb < float(CAP),
                e_rb.astype(jnp.float32) * float(CAP_PAD) + pos_rb,
                float(DROP_SENTINEL),
            )
            slot_row[:, b * BLK:(b + 1) * BLK] = slot_rf.astype(jnp.int32)
            prefr = cum_rb[:, BLK - 1:BLK]

        row_ids = base + lax.broadcasted_iota(
            jnp.int32, (ROWS_PER_DEV, n_tokens), 0
        )
        disp = (slot_row[...] == row_ids).astype(jnp.bfloat16)
        xd[...] = jnp.dot(
            disp, x_ref[...].astype(jnp.bfloat16),
            preferred_element_type=jnp.float32,
        )

        for e in range(E_LOCAL):
            w_copy(e).wait()
            res = jnp.dot(
                xd[e * CAP_PAD:(e + 1) * CAP_PAD],
                wbuf[e % N_WBUF],
                preferred_element_type=jnp.float32,
            )
            table[pl.ds(base + e * CAP_PAD, CAP_PAD), :] = res.astype(
                jnp.bfloat16
            )
            if e + N_WBUF < E_LOCAL:
                w_copy(e + N_WBUF).start()
            if pipe:
                if e < N_SUB:
                    mk_cw(0, my, e).start()
                else:
                    mk_ccw(0, my, e - N_SUB).start()

        my_slot = slot_col[pl.ds(my * tok_per_dev, tok_per_dev), :]
        chunk_col_ids = lax.broadcasted_iota(
            jnp.int32, (tok_per_dev, ROWS_PER_DEV), 1
        )

        def combine(c, first):
            comb = (my_slot == c * ROWS_PER_DEV + chunk_col_ids).astype(
                jnp.bfloat16
            )
            part = jnp.dot(
                comb,
                table[pl.ds(c * ROWS_PER_DEV, ROWS_PER_DEV), :],
                preferred_element_type=jnp.float32,
            )
            out_ref[...] = part if first else out_ref[...] + part

        if pipe:
            combine(my, first=True)
            for h in range(1, N_DEV - 1):
                c_cw = lax.rem(my - h + N_DEV, N_DEV)
                c_ccw = lax.rem(my + h, N_DEV)
                for s in range(N_SUB):
                    mk_cw(h - 1, c_cw, s).wait_recv()
                    mk_cw(h, c_cw, s).start()
                    mk_ccw(h - 1, c_ccw, s).wait_recv()
                    mk_ccw(h, c_ccw, s).start()
                if h == N_DEV - 2:
                    combine(lax.rem(my + 2, N_DEV), first=False)
            h_last = N_DEV - 2
            c_cw = lax.rem(my + 1, N_DEV)
            c_ccw = lax.rem(my - 1 + N_DEV, N_DEV)
            for s in range(N_SUB):
                mk_cw(h_last, c_cw, s).wait_recv()
                mk_ccw(h_last, c_ccw, s).wait_recv()
            combine(c_cw, first=False)
            combine(c_ccw, first=False)
            for h in range(N_DEV - 1):
                c_cw = lax.rem(my - h + N_DEV, N_DEV)
                c_ccw = lax.rem(my + h, N_DEV)
                for s in range(N_SUB):
                    mk_cw(h, c_cw, s).wait_send()
                    mk_ccw(h, c_ccw, s).wait_send()
        else:
            combine(my, first=True)
            for d in range(1, N_DEV):
                combine(lax.rem(my + d, N_DEV), first=False)

    return pl.pallas_call(
        body,
        out_shape=jax.ShapeDtypeStruct((tok_per_dev, h_out), jnp.float32),
        in_specs=[
            pl.BlockSpec(memory_space=pltpu.VMEM),
            pl.BlockSpec(memory_space=pltpu.VMEM),
            pl.BlockSpec(memory_space=pltpu.VMEM),
            pl.BlockSpec(memory_space=pltpu.MemorySpace.HBM),
        ],
        out_specs=pl.BlockSpec(memory_space=pltpu.VMEM),
        scratch_shapes=[
            pltpu.VMEM((N_ROWS, h_out), jnp.bfloat16),
            pltpu.VMEM((ROWS_PER_DEV, d_model), jnp.float32),
            pltpu.VMEM((n_tokens, 1), jnp.int32),
            pltpu.VMEM((1, n_tokens), jnp.int32),
            pltpu.VMEM((N_WBUF, d_model, h_out), jnp.float32),
            pltpu.SemaphoreType.DMA((N_WBUF,)),
            pltpu.SemaphoreType.DMA((N_DEV - 1, N_SUB)),
            pltpu.SemaphoreType.DMA((N_DEV - 1, N_SUB)),
            pltpu.SemaphoreType.DMA((N_DEV - 1, N_SUB)),
            pltpu.SemaphoreType.DMA((N_DEV - 1, N_SUB)),
        ],
        compiler_params=pltpu.CompilerParams(
            collective_id=0,
            vmem_limit_bytes=100 * 1024 * 1024,
        ),
    )(x, rt_col, rt_row, expert_W)


def kernel(x, router_W, route_idx, expert_W):
    n_tokens, _ = x.shape
    del router_W
    tok_per_dev = n_tokens // N_DEV
    rt_row = jnp.transpose(route_idx)
    return _moe_pallas(x, route_idx, rt_row, expert_W, tok_per_dev)
